# Initial kernel scaffold; baseline (speedup 1.0000x reference)
import jax
import jax.numpy as jnp
from jax.experimental import pallas as pl


def kernel(x, edge_index, edge_attr, We, be, lin_e_w, lin_e_b, w1, b1, w2, b2, ln_g, ln_b):
    raise NotImplementedError("write your pallas kernel here")


# speedup vs baseline: 1.8244x; 1.8244x over previous
import functools

import jax
import jax.numpy as jnp
from jax import lax
from jax.experimental import pallas as pl
from jax.experimental.pallas import tpu as pltpu
from jax.experimental.pallas import tpu_sc as plsc

_N = 10000
_E = 320000
_C = 128
_DE = 16
_L = 3

_TILES = 16
_NW = 2 * _TILES
_K = 128
_EPW = 10240
_EP = _EPW * _NW
_NP = 10112
_RPT = _NP // _TILES


def _edge_proj(ea_p, We, be2, lew, leb):
    Eb = 2048

    def body(ea_ref, we_ref, be_ref, lw_ref, lb_ref, o0, o1, o2):
        t = jnp.dot(ea_ref[...], we_ref[...],
                    preferred_element_type=jnp.float32) + be_ref[...]
        t = t * jax.nn.sigmoid(t)
        outs = (o0, o1, o2)
        for l in range(_L):
            outs[l][...] = jnp.dot(t, lw_ref[l],
                                   preferred_element_type=jnp.float32) + lb_ref[l]

    return pl.pallas_call(
        body,
        grid=(_EP // Eb,),
        in_specs=[
            pl.BlockSpec((Eb, _DE), lambda i: (i, 0)),
            pl.BlockSpec((_DE, _C), lambda i: (0, 0)),
            pl.BlockSpec((1, _C), lambda i: (0, 0)),
            pl.BlockSpec((_L, _C, _C), lambda i: (0, 0, 0)),
            pl.BlockSpec((_L, 1, _C), lambda i: (0, 0, 0)),
        ],
        out_specs=[pl.BlockSpec((Eb, _C), lambda i: (i, 0))] * _L,
        out_shape=[jax.ShapeDtypeStruct((_EP, _C), jnp.float32)] * _L,
    )(ea_p, We, be2, lew, leb)


def _sc_layer(ep, src, dst, x):
    mesh = plsc.VectorSubcoreMesh(core_axis_name="c", subcore_axis_name="s",
                                  num_cores=2, num_subcores=_TILES)

    @functools.partial(
        pl.kernel,
        out_type=jax.ShapeDtypeStruct((2, _NP, _C), jnp.float32),
        mesh=mesh,
        scratch_types=[
            pltpu.VMEM_SHARED((_NP, _C), jnp.float32),
            pltpu.VMEM((_K, _C), jnp.float32),
            pltpu.VMEM((_K, _C), jnp.float32),
            pltpu.VMEM((_K,), jnp.int32),
            pltpu.VMEM((1, 128), jnp.int32),
            pltpu.SemaphoreType.DMA,
        ],
    )
    def k(ep_h, src_h, dst_h, x_h, out_h, agg, epb, xgb, srcb, dstb, sem):
        c = lax.axis_index("c")
        s = lax.axis_index("s")
        r0 = s * _RPT

        @pl.loop(0, _K)
        def _z(r):
            for cb in range(_C // 16):
                epb[r, pl.ds(cb * 16, 16)] = jnp.zeros((16,), jnp.float32)

        for t in range(_RPT // _K):
            pltpu.sync_copy(epb, agg.at[pl.ds(r0 + t * _K, _K)])
        _rem = _RPT % _K
        pltpu.sync_copy(epb.at[pl.ds(0, _rem)],
                        agg.at[pl.ds(r0 + _RPT - _rem, _rem)])
        plsc.subcore_barrier()

        e_base = (c * _TILES + s) * _EPW

        @pl.loop(0, _EPW // _K)
        def _chunk(g):
            e0 = e_base + g * _K
            pltpu.sync_copy(src_h.at[pl.ds(e0, _K)], srcb)
            pltpu.sync_copy(dst_h.at[pl.ds(e0, 128)], dstb.at[0])
            pltpu.sync_copy(ep_h.at[pl.ds(e0, _K)], epb)
            pltpu.async_copy(x_h.at[srcb], xgb, sem).wait()

            @pl.loop(0, _K)
            def _row(r):
                for cb in range(_C // 16):
                    sl = pl.ds(cb * 16, 16)
                    xgb[r, sl] = jnp.maximum(xgb[r, sl] + epb[r, sl], 0.0)

            pltpu.sync_copy(xgb, agg.at[dstb.at[0]], add=True)

        plsc.subcore_barrier()
        pltpu.sync_copy(agg.at[pl.ds(r0, _RPT)],
                        out_h.at[c, pl.ds(r0, _RPT)])

    return k(ep, src, dst, x)


def _node_mlp(hp2, x, w1l, b1l, w2l, b2l, gl, btl):
    Nb = _RPT

    def body(hp_ref, x_ref, w1_ref, b1_ref, w2_ref, b2_ref, g_ref, bt_ref, o_ref):
        xv = x_ref[...]
        hp = hp_ref[0] + hp_ref[1] + xv
        t = jnp.dot(hp, w1_ref[...],
                    preferred_element_type=jnp.float32) + b1_ref[...]
        t = t * jax.nn.sigmoid(t)
        h = jnp.dot(t, w2_ref[...],
                    preferred_element_type=jnp.float32) + b2_ref[...]
        y = xv + h
        mu = jnp.mean(y, axis=-1, keepdims=True)
        d = y - mu
        var = jnp.mean(d * d, axis=-1, keepdims=True)
        o_ref[...] = d * lax.rsqrt(var + 1e-5) * g_ref[...] + bt_ref[...]

    full = lambda i: (0, 0)
    return pl.pallas_call(
        body,
        grid=(_NP // Nb,),
        in_specs=[
            pl.BlockSpec((2, Nb, _C), lambda i: (0, i, 0)),
            pl.BlockSpec((Nb, _C), lambda i: (i, 0)),
            pl.BlockSpec((_C, _C), full),
            pl.BlockSpec((1, _C), full),
            pl.BlockSpec((_C, _C), full),
            pl.BlockSpec((1, _C), full),
            pl.BlockSpec((1, _C), full),
            pl.BlockSpec((1, _C), full),
        ],
        out_specs=pl.BlockSpec((Nb, _C), lambda i: (i, 0)),
        out_shape=jax.ShapeDtypeStruct((_NP, _C), jnp.float32),
    )(hp2, x, w1l, b1l, w2l, b2l, gl, btl)


def kernel(x, edge_index, edge_attr, We, be, lin_e_w, lin_e_b, w1, b1, w2, b2,
           ln_g, ln_b):
    pad = _EP - _E
    src = jnp.pad(edge_index[0], (0, pad))
    dst = jnp.pad(edge_index[1], (0, pad), constant_values=_N)
    ea_p = jnp.pad(edge_attr, ((0, pad), (0, 0)))
    eps = _edge_proj(ea_p, We, be.reshape(1, _C), lin_e_w,
                     lin_e_b.reshape(_L, 1, _C))
    xp = jnp.pad(x, ((0, _NP - _N), (0, 0)))
    for l in range(_L):
        hp2 = _sc_layer(eps[l], src, dst, xp)
        xp = _node_mlp(hp2, xp, w1[l], b1[l].reshape(1, _C), w2[l],
                       b2[l].reshape(1, _C), ln_g[l].reshape(1, _C),
                       ln_b[l].reshape(1, _C))
    return xp[:_N]


# speedup vs baseline: 2.5237x; 1.3832x over previous
import functools

import jax
import jax.numpy as jnp
from jax import lax
from jax.experimental import pallas as pl
from jax.experimental.pallas import tpu as pltpu
from jax.experimental.pallas import tpu_sc as plsc

_N = 10000
_E = 320000
_C = 128
_DE = 16
_L = 3

_TILES = 16
_NW = 2 * _TILES
_K = 80
_EPW = 10240
_EP = _EPW * _NW
_NP = 10112
_RPT = _NP // _TILES


def _edge_proj(ea_p, We, be2, lew, leb):
    Eb = 2048

    def body(ea_ref, we_ref, be_ref, lw_ref, lb_ref, o0, o1, o2):
        t = jnp.dot(ea_ref[...], we_ref[...],
                    preferred_element_type=jnp.float32) + be_ref[...]
        t = t * jax.nn.sigmoid(t)
        outs = (o0, o1, o2)
        for l in range(_L):
            outs[l][...] = jnp.dot(t, lw_ref[l],
                                   preferred_element_type=jnp.float32) + lb_ref[l]

    return pl.pallas_call(
        body,
        grid=(_EP // Eb,),
        in_specs=[
            pl.BlockSpec((Eb, _DE), lambda i: (i, 0)),
            pl.BlockSpec((_DE, _C), lambda i: (0, 0)),
            pl.BlockSpec((1, _C), lambda i: (0, 0)),
            pl.BlockSpec((_L, _C, _C), lambda i: (0, 0, 0)),
            pl.BlockSpec((_L, 1, _C), lambda i: (0, 0, 0)),
        ],
        out_specs=[pl.BlockSpec((Eb, _C), lambda i: (i, 0))] * _L,
        out_shape=[jax.ShapeDtypeStruct((_EP, _C), jnp.float32)] * _L,
    )(ea_p, We, be2, lew, leb)


def _sc_layer(ep, src, dst, x):
    mesh = plsc.VectorSubcoreMesh(core_axis_name="c", subcore_axis_name="s",
                                  num_cores=2, num_subcores=_TILES)

    @functools.partial(
        pl.kernel,
        out_type=jax.ShapeDtypeStruct((2, _NP, _C), jnp.float32),
        mesh=mesh,
        scratch_types=[
            pltpu.VMEM_SHARED((_NP, _C), jnp.float32),
            pltpu.VMEM((2, _K, _C), jnp.float32),
            pltpu.VMEM((2, _K, _C), jnp.float32),
            pltpu.VMEM((4, _K), jnp.int32),
            pltpu.VMEM((4, _K), jnp.int32),
            pltpu.SemaphoreType.DMA((2,)),
            pltpu.SemaphoreType.DMA((4,)),
            pltpu.SemaphoreType.DMA((2,)),
            pltpu.SemaphoreType.DMA((2,)),
        ],
    )
    def k(ep_h, src_h, dst_h, x_h, out_h, agg, epb, xgb, srcb, dstb,
          sem_ep, sem_ix, sem_g, sem_sc):
        c = lax.axis_index("c")
        s = lax.axis_index("s")
        r0 = s * _RPT
        nck = _EPW // _K
        e_base = (c * _TILES + s) * _EPW

        @pl.loop(0, _K)
        def _z(r):
            for cb in range(_C // 16):
                epb[0, r, pl.ds(cb * 16, 16)] = jnp.zeros((16,), jnp.float32)

        for t in range(_RPT // _K):
            pltpu.sync_copy(epb.at[0], agg.at[pl.ds(r0 + t * _K, _K)])
        _rem = _RPT % _K
        pltpu.sync_copy(epb.at[0, pl.ds(0, _rem)],
                        agg.at[pl.ds(r0 + _RPT - _rem, _rem)])
        plsc.subcore_barrier()

        def fetch(g, b2, b4):
            e0 = e_base + g * _K
            pltpu.async_copy(src_h.at[pl.ds(e0, _K)], srcb.at[b4],
                             sem_ix.at[b4])
            pltpu.async_copy(dst_h.at[pl.ds(e0, _K)], dstb.at[b4],
                             sem_ix.at[b4])
            pltpu.async_copy(ep_h.at[pl.ds(e0, _K)], epb.at[b2],
                             sem_ep.at[b2])

        def wait_idx(b4):
            pltpu.make_async_copy(src_h.at[pl.ds(0, _K)], srcb.at[b4],
                                  sem_ix.at[b4]).wait()
            pltpu.make_async_copy(dst_h.at[pl.ds(0, _K)], dstb.at[b4],
                                  sem_ix.at[b4]).wait()

        def gather(b2, b4):
            pltpu.async_copy(x_h.at[srcb.at[b4]], xgb.at[b2], sem_g.at[b2])

        def wait_gather(b2, b4):
            pltpu.make_async_copy(x_h.at[srcb.at[b4]], xgb.at[b2],
                                  sem_g.at[b2]).wait()

        def wait_ep(b2):
            pltpu.make_async_copy(ep_h.at[pl.ds(0, _K)], epb.at[b2],
                                  sem_ep.at[b2]).wait()

        def scatter(b2, b4):
            pltpu.async_copy(xgb.at[b2], agg.at[dstb.at[b4]], sem_sc.at[b2],
                             add=True)

        def wait_scatter(b2, b4):
            pltpu.make_async_copy(xgb.at[b2], agg.at[dstb.at[b4]],
                                  sem_sc.at[b2]).wait()

        def compute(b2):
            @pl.loop(0, _K)
            def _row(r):
                for cb in range(_C // 16):
                    sl = pl.ds(cb * 16, 16)
                    xgb[b2, r, sl] = jnp.maximum(
                        xgb[b2, r, sl] + epb[b2, r, sl], 0.0)

        fetch(0, 0, 0)
        wait_idx(0)
        gather(0, 0)
        fetch(1, 1, 1)

        @pl.loop(0, nck // 4)
        def _outer(g4):
            for u in range(4):
                b2 = u % 2
                nb2 = (u + 1) % 2
                g = g4 * 4 + u

                @pl.when(g + 1 < nck)
                def _():
                    wait_idx((u + 1) % 4)

                    @pl.when(g >= 1)
                    def _():
                        wait_scatter(nb2, (u + 3) % 4)

                    gather(nb2, (u + 1) % 4)

                wait_ep(b2)
                wait_gather(b2, u % 4)
                compute(b2)
                scatter(b2, u % 4)

                @pl.when(g + 2 < nck)
                def _():
                    fetch(g + 2, b2, (u + 2) % 4)

        wait_scatter(0, (nck - 2) % 4)
        wait_scatter(1, (nck - 1) % 4)
        plsc.subcore_barrier()
        pltpu.sync_copy(agg.at[pl.ds(r0, _RPT)],
                        out_h.at[c, pl.ds(r0, _RPT)])

    return k(ep, src, dst, x)


def _node_mlp(hp2, x, w1l, b1l, w2l, b2l, gl, btl):
    Nb = _RPT

    def body(hp_ref, x_ref, w1_ref, b1_ref, w2_ref, b2_ref, g_ref, bt_ref, o_ref):
        xv = x_ref[...]
        hp = hp_ref[0] + hp_ref[1] + xv
        t = jnp.dot(hp, w1_ref[...],
                    preferred_element_type=jnp.float32) + b1_ref[...]
        t = t * jax.nn.sigmoid(t)
        h = jnp.dot(t, w2_ref[...],
                    preferred_element_type=jnp.float32) + b2_ref[...]
        y = xv + h
        mu = jnp.mean(y, axis=-1, keepdims=True)
        d = y - mu
        var = jnp.mean(d * d, axis=-1, keepdims=True)
        o_ref[...] = d * lax.rsqrt(var + 1e-5) * g_ref[...] + bt_ref[...]

    full = lambda i: (0, 0)
    return pl.pallas_call(
        body,
        grid=(_NP // Nb,),
        in_specs=[
            pl.BlockSpec((2, Nb, _C), lambda i: (0, i, 0)),
            pl.BlockSpec((Nb, _C), lambda i: (i, 0)),
            pl.BlockSpec((_C, _C), full),
            pl.BlockSpec((1, _C), full),
            pl.BlockSpec((_C, _C), full),
            pl.BlockSpec((1, _C), full),
            pl.BlockSpec((1, _C), full),
            pl.BlockSpec((1, _C), full),
        ],
        out_specs=pl.BlockSpec((Nb, _C), lambda i: (i, 0)),
        out_shape=jax.ShapeDtypeStruct((_NP, _C), jnp.float32),
    )(hp2, x, w1l, b1l, w2l, b2l, gl, btl)


def kernel(x, edge_index, edge_attr, We, be, lin_e_w, lin_e_b, w1, b1, w2, b2,
           ln_g, ln_b):
    pad = _EP - _E
    src = jnp.pad(edge_index[0], (0, pad))
    dst = jnp.pad(edge_index[1], (0, pad), constant_values=_N)
    ea_p = jnp.pad(edge_attr, ((0, pad), (0, 0)))
    eps = _edge_proj(ea_p, We, be.reshape(1, _C), lin_e_w,
                     lin_e_b.reshape(_L, 1, _C))
    xp = jnp.pad(x, ((0, _NP - _N), (0, 0)))
    for l in range(_L):
        hp2 = _sc_layer(eps[l], src, dst, xp)
        xp = _node_mlp(hp2, xp, w1[l], b1[l].reshape(1, _C), w2[l],
                       b2[l].reshape(1, _C), ln_g[l].reshape(1, _C),
                       ln_b[l].reshape(1, _C))
    return xp[:_N]
